# Initial kernel scaffold; baseline (speedup 1.0000x reference)
import jax
import jax.numpy as jnp
from jax.experimental import pallas as pl


def kernel(x_paper, x_author, edge_index_pa, edge_index_ap, W_kqv_paper, b_kqv_paper, W_kqv_author, b_kqv_author, W_out_paper, b_out_paper, W_out_author, b_out_author, W_krel, b_krel, W_vrel, b_vrel, ln_w_paper, ln_b_paper, ln_w_author, ln_b_author, skip_paper, skip_author, p_rel_pa, p_rel_ap):
    raise NotImplementedError("write your pallas kernel here")


# speedup vs baseline: 20.7062x; 20.7062x over previous
import functools

import jax
import jax.numpy as jnp
from jax import lax
from jax.experimental import pallas as pl
from jax.experimental.pallas import tpu as pltpu
from jax.experimental.pallas import tpu_sc as plsc

H = 8
D = 16
C = 128
NP_ = 50000
NA_ = 50000

WN = 256
EB = 512
KMAX = 8
ND_PAD = 100352
EP_PAD = 602112
NWIN = ND_PAD // WN
NEB = EP_PAD // EB


def _proj_body(x_ref, wkqv_ref, bkqv_ref, wbk_ref, bk_ref, wbv_ref, bv_ref,
               q_ref, k_ref, v_ref):
    kqv = jnp.dot(x_ref[...], wkqv_ref[...],
                  preferred_element_type=jnp.float32) + bkqv_ref[...]
    q_ref[...] = kqv[:, C:2 * C]
    k_ref[...] = jnp.dot(kqv[:, :C], wbk_ref[...],
                         preferred_element_type=jnp.float32) + bk_ref[...]
    v_ref[...] = jnp.dot(kqv[:, 2 * C:], wbv_ref[...],
                         preferred_element_type=jnp.float32) + bv_ref[...]


def _proj(x, wkqv, bkqv, wbk, bk, wbv, bv):
    n = x.shape[0]
    blk = 2000
    grid = (n // blk,)
    full = lambda i: (0, 0)
    return pl.pallas_call(
        _proj_body,
        grid=grid,
        in_specs=[
            pl.BlockSpec((blk, C), lambda i: (i, 0)),
            pl.BlockSpec((C, 3 * C), full),
            pl.BlockSpec((1, 3 * C), full),
            pl.BlockSpec((C, C), full),
            pl.BlockSpec((1, C), full),
            pl.BlockSpec((C, C), full),
            pl.BlockSpec((1, C), full),
        ],
        out_specs=[
            pl.BlockSpec((blk, C), lambda i: (i, 0)),
            pl.BlockSpec((blk, C), lambda i: (i, 0)),
            pl.BlockSpec((blk, C), lambda i: (i, 0)),
        ],
        out_shape=[jax.ShapeDtypeStruct((n, C), jnp.float32)] * 3,
    )(x, wkqv, bkqv, wbk, bk, wbv, bv)


def _sc_gather2(k_tab, v_tab, idx):
    info = plsc.get_sparse_core_info()
    nw = info.num_cores * info.num_subcores
    ep = idx.shape[0]
    per_w = ep // nw
    ch = 128
    nch = per_w // ch
    mesh = plsc.VectorSubcoreMesh(core_axis_name="c", subcore_axis_name="s")

    @functools.partial(
        pl.kernel, mesh=mesh,
        out_type=[jax.ShapeDtypeStruct((ep, C), jnp.float32),
                  jax.ShapeDtypeStruct((ep, C), jnp.float32)],
        scratch_types=[
            pltpu.VMEM((ch,), jnp.int32),
            pltpu.VMEM((ch, C), jnp.float32),
            pltpu.VMEM((ch, C), jnp.float32),
            pltpu.SemaphoreType.DMA,
            pltpu.SemaphoreType.DMA,
        ],
    )
    def kern(k_hbm, v_hbm, idx_hbm, ko_hbm, vo_hbm, idx_v, krows, vrows,
             sk, sv):
        wid = lax.axis_index("s") * info.num_cores + lax.axis_index("c")
        base = wid * per_w

        def body(j, carry):
            off = base + j * ch
            pltpu.sync_copy(idx_hbm.at[pl.ds(off, ch)], idx_v)
            cpk = pltpu.async_copy(k_hbm.at[idx_v], krows, sk)
            cpv = pltpu.async_copy(v_hbm.at[idx_v], vrows, sv)
            cpk.wait()
            cpv.wait()
            pltpu.sync_copy(krows, ko_hbm.at[pl.ds(off, ch)])
            pltpu.sync_copy(vrows, vo_hbm.at[pl.ds(off, ch)])
            return carry

        lax.fori_loop(0, nch, body, 0)

    return kern(k_tab, v_tab, idx)


def _edge_body(cidx_ref, dst_ref, flag_ref, k_ref, v_ref, q_ref,
               prpa_ref, prap_ref, den_ref, agg_ref):
    b = pl.program_id(0)
    kk = pl.program_id(1)

    dstl = dst_ref[0, 0, :] - b * WN
    oh = (dstl[:, None] == lax.broadcasted_iota(jnp.int32, (EB, WN), 1)
          ).astype(jnp.float32)
    sel = (lax.broadcasted_iota(jnp.int32, (C, H), 0) // D ==
           lax.broadcasted_iota(jnp.int32, (C, H), 1)).astype(jnp.float32)

    kv = k_ref[...]
    vv = v_ref[...]
    qi = jnp.dot(oh, q_ref[...], preferred_element_type=jnp.float32)
    alpha = jnp.dot(qi * kv, sel, preferred_element_type=jnp.float32)
    flag = flag_ref[0, 0, :][:, None]
    attr = flag * prpa_ref[...] + (1.0 - flag) * prap_ref[...]
    ex = jnp.exp(alpha * attr * 0.25)
    msg = vv * jnp.dot(ex, sel.T, preferred_element_type=jnp.float32)

    @pl.when(kk == 0)
    def _():
        den_ref[...] = jnp.zeros_like(den_ref)
        agg_ref[...] = jnp.zeros_like(agg_ref)

    cdims = (((0,), (0,)), ((), ()))
    den_ref[...] += lax.dot_general(oh, ex, cdims,
                                    preferred_element_type=jnp.float32)
    agg_ref[...] += lax.dot_general(oh, msg, cdims,
                                    preferred_element_type=jnp.float32)


def _edge_phase(cidx, dst3, flag3, krows, vrows, qtab, prpa, prap):
    grid_spec = pltpu.PrefetchScalarGridSpec(
        num_scalar_prefetch=1,
        grid=(NWIN, KMAX),
        in_specs=[
            pl.BlockSpec((1, 1, EB), lambda b, k, c: (c[b, k], 0, 0)),
            pl.BlockSpec((1, 1, EB), lambda b, k, c: (c[b, k], 0, 0)),
            pl.BlockSpec((EB, C), lambda b, k, c: (c[b, k], 0)),
            pl.BlockSpec((EB, C), lambda b, k, c: (c[b, k], 0)),
            pl.BlockSpec((WN, C), lambda b, k, c: (b, 0)),
            pl.BlockSpec((1, H), lambda b, k, c: (0, 0)),
            pl.BlockSpec((1, H), lambda b, k, c: (0, 0)),
        ],
        out_specs=[
            pl.BlockSpec((WN, H), lambda b, k, c: (b, 0)),
            pl.BlockSpec((WN, C), lambda b, k, c: (b, 0)),
        ],
    )
    return pl.pallas_call(
        _edge_body,
        grid_spec=grid_spec,
        out_shape=[jax.ShapeDtypeStruct((ND_PAD, H), jnp.float32),
                   jax.ShapeDtypeStruct((ND_PAD, C), jnp.float32)],
    )(cidx, dst3, flag3, krows, vrows, qtab, prpa, prap)


def _finish_body(agg_ref, den_ref, x_ref, w_ref, b_ref, lnw_ref, lnb_ref,
                 sp_ref, o_ref):
    sel = (lax.broadcasted_iota(jnp.int32, (H, C), 1) // D ==
           lax.broadcasted_iota(jnp.int32, (H, C), 0)).astype(jnp.float32)
    den = jnp.dot(den_ref[...], sel,
                  preferred_element_type=jnp.float32) + 1e-16
    h = jnp.dot(agg_ref[...] / den, w_ref[...],
                preferred_element_type=jnp.float32) + b_ref[...]
    sp = sp_ref[0, 0]
    o = sp * h + (1.0 - sp) * x_ref[...]
    mu = jnp.mean(o, axis=1, keepdims=True)
    var = jnp.mean((o - mu) ** 2, axis=1, keepdims=True)
    o = (o - mu) / jnp.sqrt(var + 1e-5) * lnw_ref[...] + lnb_ref[...]
    o_ref[...] = o * 0.5 * (1.0 + lax.erf(o * 0.7071067811865475))


def _finish(agg, den, x, w, b, lnw, lnb, sp):
    n = x.shape[0]
    blk = 2000
    full = lambda i: (0, 0)
    return pl.pallas_call(
        _finish_body,
        grid=(n // blk,),
        in_specs=[
            pl.BlockSpec((blk, C), lambda i: (i, 0)),
            pl.BlockSpec((blk, H), lambda i: (i, 0)),
            pl.BlockSpec((blk, C), lambda i: (i, 0)),
            pl.BlockSpec((C, C), full),
            pl.BlockSpec((1, C), full),
            pl.BlockSpec((1, C), full),
            pl.BlockSpec((1, C), full),
            pl.BlockSpec((1, 1), full),
        ],
        out_specs=pl.BlockSpec((blk, C), lambda i: (i, 0)),
        out_shape=jax.ShapeDtypeStruct((n, C), jnp.float32),
    )(agg, den, x, w, b, lnw, lnb, sp)


def _blockdiag(w):
    out = jnp.zeros((C, C), jnp.float32)
    for h in range(H):
        out = out.at[h * D:(h + 1) * D, h * D:(h + 1) * D].set(w[h])
    return out


def kernel(x_paper, x_author, edge_index_pa, edge_index_ap, W_kqv_paper,
           b_kqv_paper, W_kqv_author, b_kqv_author, W_out_paper, b_out_paper,
           W_out_author, b_out_author, W_krel, b_krel, W_vrel, b_vrel,
           ln_w_paper, ln_b_paper, ln_w_author, ln_b_author, skip_paper,
           skip_author, p_rel_pa, p_rel_ap):
    epa = edge_index_pa.shape[1]
    eap = edge_index_ap.shape[1]

    idx0 = jnp.arange(H) * 2
    wbk_p = _blockdiag(W_krel[idx0])
    wbk_a = _blockdiag(W_krel[idx0 + 1])
    wbv_p = _blockdiag(W_vrel[idx0])
    wbv_a = _blockdiag(W_vrel[idx0 + 1])
    bk_p = b_krel[idx0].reshape(1, C)
    bk_a = b_krel[idx0 + 1].reshape(1, C)
    bv_p = b_vrel[idx0].reshape(1, C)
    bv_a = b_vrel[idx0 + 1].reshape(1, C)

    q_p, k_p, v_p = _proj(x_paper, W_kqv_paper, b_kqv_paper.reshape(1, -1),
                          wbk_p, bk_p, wbv_p, bv_p)
    q_a, k_a, v_a = _proj(x_author, W_kqv_author, b_kqv_author.reshape(1, -1),
                          wbk_a, bk_a, wbv_a, bv_a)

    qtab = jnp.concatenate(
        [q_p, q_a, jnp.zeros((ND_PAD - NP_ - NA_, C), jnp.float32)], axis=0)
    ktab = jnp.concatenate([k_p, k_a], axis=0)
    vtab = jnp.concatenate([v_p, v_a], axis=0)

    src = jnp.concatenate([edge_index_pa[0],
                           edge_index_ap[0] + NP_]).astype(jnp.int32)
    dst = jnp.concatenate([edge_index_pa[1] + NP_,
                           edge_index_ap[1]]).astype(jnp.int32)
    perm = jnp.argsort(dst)
    npad = EP_PAD - epa - eap
    src_s = jnp.concatenate([src[perm], jnp.zeros((npad,), jnp.int32)])
    dst_s = jnp.concatenate([dst[perm],
                             jnp.full((npad,), ND_PAD - 1, jnp.int32)])
    flag_s = jnp.concatenate([(perm < epa).astype(jnp.float32),
                              jnp.zeros((npad,), jnp.float32)])

    starts = jnp.searchsorted(dst_s, jnp.arange(NWIN, dtype=jnp.int32) * WN)
    cidx = jnp.clip(starts[:, None] // EB + jnp.arange(KMAX)[None, :],
                    0, NEB - 1).astype(jnp.int32)

    krows, vrows = _sc_gather2(ktab, vtab, src_s)

    den, agg = _edge_phase(cidx, dst_s.reshape(NEB, 1, EB),
                           flag_s.reshape(NEB, 1, EB), krows, vrows, qtab,
                           p_rel_pa, p_rel_ap)

    sp = jax.nn.sigmoid(skip_paper).reshape(1, 1)
    sa = jax.nn.sigmoid(skip_author).reshape(1, 1)
    o_p = _finish(agg[:NP_], den[:NP_], x_paper, W_out_paper,
                  b_out_paper.reshape(1, -1), ln_w_paper.reshape(1, -1),
                  ln_b_paper.reshape(1, -1), sp)
    o_a = _finish(agg[NP_:NP_ + NA_], den[NP_:NP_ + NA_], x_author,
                  W_out_author, b_out_author.reshape(1, -1),
                  ln_w_author.reshape(1, -1), ln_b_author.reshape(1, -1), sa)
    return jnp.concatenate([o_p, o_a], axis=0)
